# baseline (speedup 1.0000x reference)
import functools

import jax
import jax.numpy as jnp
from jax import lax
from jax.experimental import pallas as pl
from jax.experimental.pallas import tpu as pltpu
from jax.experimental.pallas import tpu_sc as plsc

_VOCAB = 1000000
_MOD = _VOCAB - 1
_D = 64
_B = 4096
_S = 200
_SPAD = 208
_NCHUNK = _SPAD // 16
_GCH = 104

_NC, _NS = 2, 16
_NW = _NC * _NS
_ROWS_PER_W = _B // _NW


def _floor_mod(x):
    r = lax.rem(x, jnp.int32(_MOD))
    return jnp.where(r < 0, r + jnp.int32(_MOD), r)


@functools.partial(
    pl.kernel,
    out_type=jax.ShapeDtypeStruct((_B * _S, _D), jnp.float32),
    mesh=plsc.VectorSubcoreMesh(core_axis_name="c", subcore_axis_name="s"),
    compiler_params=pltpu.CompilerParams(use_tc_tiling_on_sc=False),
    scratch_types=[
        pltpu.VMEM((224,), jnp.int32),
        pltpu.VMEM((_SPAD,), jnp.int32),
        pltpu.VMEM((_SPAD, _D), jnp.float32),
        pltpu.VMEM((16,), jnp.float32),
        pltpu.SemaphoreType.DMA,
    ],
)
def _sc_embed(tok_hbm, table_hbm, scale_hbm, out_hbm,
              tok_v, idx_v, rows_v, scale_v, sem):
    wid = lax.axis_index("s") * _NC + lax.axis_index("c")
    base_row = wid * _ROWS_PER_W

    pltpu.sync_copy(scale_hbm, scale_v)
    sval = scale_v[...]

    tok_v[pl.ds(0, 16)] = jnp.zeros((16,), jnp.int32)

    def row_body(r, carry):
        row = base_row + r
        pltpu.sync_copy(tok_hbm.at[pl.ds(row * _S, _S)], tok_v.at[pl.ds(8, _S)])

        for j in range(_NCHUNK):
            a = tok_v[pl.ds(8 + 16 * j, 16)]
            b = tok_v[pl.ds(7 + 16 * j, 16)]
            c = tok_v[pl.ds(6 + 16 * j, 16)]
            h = _floor_mod((a * jnp.int32(36313))
                           ^ (b * jnp.int32(27191))
                           ^ (c * jnp.int32(51647)))
            if j == 0:
                h = jnp.where(lax.iota(jnp.int32, 16) == 0,
                              jnp.int32(_MOD), h)
            idx_v[pl.ds(16 * j, 16)] = h

        cps = []
        for g in range(_SPAD // _GCH):
            cps.append(pltpu.async_copy(
                table_hbm.at[idx_v.at[pl.ds(g * _GCH, _GCH)]],
                rows_v.at[pl.ds(g * _GCH, _GCH)],
                sem))
        for cp in cps:
            cp.wait()

        def scale_body(i, carry2):
            for k in range(_D // 16):
                rows_v[i, pl.ds(16 * k, 16)] = (
                    rows_v[i, pl.ds(16 * k, 16)] * sval)
            return carry2

        lax.fori_loop(0, _S, scale_body, 0, unroll=2)

        pltpu.sync_copy(rows_v.at[pl.ds(0, _S)],
                        out_hbm.at[pl.ds(row * _S, _S)])
        return carry

    lax.fori_loop(0, _ROWS_PER_W, row_body, 0)


def kernel(token_ids, embed_table, scale):
    scale_vec = jnp.full((16,), scale, dtype=jnp.float32)
    out = _sc_embed(token_ids.reshape(-1), embed_table, scale_vec)
    return out.reshape(_B, _S, _D)
